# baseline (speedup 1.0000x reference)
import functools

import jax
import jax.numpy as jnp
from jax import lax
from jax.experimental import pallas as pl
from jax.experimental.pallas import tpu as pltpu
from jax.experimental.pallas import tpu_sc as plsc

EMB_DIM = 64
NBUF = 8

_info = plsc.get_sparse_core_info()
NC, NS = _info.num_cores, _info.num_subcores
NW = NC * NS


def _make_gather(n_rows, n_cols):
    r_per_w = n_rows // NW
    n_outer = r_per_w // NBUF
    mesh = plsc.VectorSubcoreMesh(core_axis_name="c", subcore_axis_name="s")

    @functools.partial(
        pl.kernel,
        mesh=mesh,
        out_type=jax.ShapeDtypeStruct((n_rows, n_cols, EMB_DIM), jnp.float32),
        compiler_params=pltpu.CompilerParams(use_tc_tiling_on_sc=False),
        scratch_types=[
            pltpu.VMEM((r_per_w, n_cols), jnp.int32),
            pltpu.VMEM((NBUF, n_cols, EMB_DIM), jnp.float32),
        ]
        + [pltpu.SemaphoreType.DMA] * (2 * NBUF),
    )
    def gather_kernel(table_hbm, idx_hbm, out_hbm, idx_v, bufs, *sems):
        gsem = sems[:NBUF]
        wsem = sems[NBUF:]
        wid = lax.axis_index("s") * NC + lax.axis_index("c")
        base = wid * r_per_w
        pltpu.sync_copy(idx_hbm.at[pl.ds(base, r_per_w)], idx_v)

        def gather(j, b):
            return pltpu.make_async_copy(
                table_hbm.at[idx_v.at[j]], bufs.at[b], gsem[b])

        def write(j, b):
            return pltpu.make_async_copy(
                bufs.at[b], out_hbm.at[base + j], wsem[b])

        for b in range(NBUF):
            gather(b, b).start()

        def outer(g, carry):
            for b in range(NBUF):
                j = g * NBUF + b
                gather(j, b).wait()
                write(j, b).start()
                nxt = j + NBUF

                @pl.when(nxt < r_per_w)
                def _():
                    write(j, b).wait()
                    gather(nxt, b).start()

            return carry

        lax.fori_loop(0, n_outer, outer, 0)
        for b in range(NBUF):
            write(r_per_w - NBUF + b, b).wait()

    return gather_kernel


_gather = _make_gather(4096, 50)


def kernel(x, table):
    return _gather(table, x)


# speedup vs baseline: 1.0617x; 1.0617x over previous
import functools

import jax
import jax.numpy as jnp
from jax import lax
from jax.experimental import pallas as pl
from jax.experimental.pallas import tpu as pltpu
from jax.experimental.pallas import tpu_sc as plsc

EMB_DIM = 64
WIDE = 2 * EMB_DIM
NBUF = 8

_info = plsc.get_sparse_core_info()
NC, NS = _info.num_cores, _info.num_subcores
NW = NC * NS


def _make_gather(n_rows, n_cols):
    r_per_w = n_rows // NW
    n_outer = r_per_w // NBUF
    mesh = plsc.VectorSubcoreMesh(core_axis_name="c", subcore_axis_name="s")

    @functools.partial(
        pl.kernel,
        mesh=mesh,
        out_type=jax.ShapeDtypeStruct((n_rows, n_cols, EMB_DIM), jnp.float32),
        compiler_params=pltpu.CompilerParams(use_tc_tiling_on_sc=False),
        scratch_types=[
            pltpu.VMEM((r_per_w, n_cols), jnp.int32),
            pltpu.VMEM((NBUF, n_cols, WIDE), jnp.float32),
        ]
        + [pltpu.SemaphoreType.DMA] * (2 * NBUF),
    )
    def gather_kernel(table_hbm, idx_hbm, out_hbm, idx_v, bufs, *sems):
        gsem = sems[:NBUF]
        wsem = sems[NBUF:]
        wid = lax.axis_index("s") * NC + lax.axis_index("c")
        base = wid * r_per_w
        pltpu.sync_copy(idx_hbm.at[pl.ds(base, r_per_w)], idx_v)

        def gather(j, b):
            return pltpu.make_async_copy(
                table_hbm.at[idx_v.at[j]], bufs.at[b], gsem[b])

        def write(j, b):
            return pltpu.make_async_copy(
                bufs.at[b, :, pl.ds(0, EMB_DIM)], out_hbm.at[base + j], wsem[b])

        for b in range(NBUF):
            gather(b, b).start()

        def outer(g, carry):
            for b in range(NBUF):
                j = g * NBUF + b
                gather(j, b).wait()
                write(j, b).start()
                nxt = j + NBUF

                @pl.when(nxt < r_per_w)
                def _():
                    write(j, b).wait()
                    gather(nxt, b).start()

            return carry

        lax.fori_loop(0, n_outer, outer, 0)
        for b in range(NBUF):
            write(r_per_w - NBUF + b, b).wait()

    return gather_kernel


_gather = _make_gather(4096, 50)


def kernel(x, table):
    t128 = jnp.pad(table, ((0, 0), (0, WIDE - EMB_DIM)))
    return _gather(t128, x)
